# Initial kernel scaffold; baseline (speedup 1.0000x reference)
import jax
import jax.numpy as jnp
from jax.experimental import pallas as pl


def kernel(features_omics1, features_omics2, adj_spatial, adj_feature_omics1, adj_feature_omics2, W_enc1, W_enc2, fc_mu1_w, fc_mu1_b, fc_logvar1_w, fc_logvar1_b, fc_mu2_w, fc_mu2_b, fc_logvar2_w, fc_logvar2_b, mu_w, mu_b, logvar_w, logvar_b, ib_w, ib_b, fus_w, fus_b, W_dec1, W_dec2):
    raise NotImplementedError("write your pallas kernel here")


# speedup vs baseline: 6.2981x; 6.2981x over previous
import functools

import jax
import jax.numpy as jnp
from jax import lax
from jax.experimental import pallas as pl
from jax.experimental.pallas import tpu as pltpu
from jax.experimental.pallas import tpu_sc as plsc

_N = 10000
_D = 128
_E = 320000
_NC = 2
_NS = 16
_SLAB = 128
_GROUP = 8
_N_ACC = 10240
_EROWS = 2560
_E_PAD = _EROWS * _SLAB


def _sc_mesh():
    return plsc.VectorSubcoreMesh(core_axis_name="c", subcore_axis_name="s",
                                  num_cores=_NC, num_subcores=_NS)


def _sc_spmm_wide(xs, src2, dst, z128):
    @functools.partial(
        pl.kernel,
        out_type=jax.ShapeDtypeStruct((_NC * _N_ACC, _D), jnp.float32),
        mesh=_sc_mesh(),
        scratch_types=[
            pltpu.VMEM((_GROUP, _SLAB), jnp.int32),
            pltpu.VMEM((_GROUP, _SLAB), jnp.int32),
            pltpu.VMEM((_SLAB, _D), jnp.float32),
            pltpu.VMEM((80, _D), jnp.float32),
            pltpu.VMEM_SHARED((_N_ACC, _D), jnp.float32),
            pltpu.SemaphoreType.DMA,
        ],
    )
    def k(xs_hbm, src_hbm, dst_hbm, z_hbm, out_hbm,
          src_v, dst_v, rows_v, bnc_v, acc_sh, sem):
        c = lax.axis_index("c")
        s = lax.axis_index("s")
        pltpu.sync_copy(z_hbm, bnc_v)
        for k8 in range(8):
            pltpu.sync_copy(bnc_v, acc_sh.at[pl.ds(s * 640 + k8 * 80, 80)])
        plsc.subcore_barrier()

        slabs_per_tile = _EROWS // _NS

        def group(g, carry):
            r0 = s * slabs_per_tile + g * _GROUP
            pltpu.sync_copy(src_hbm.at[pl.ds(c * _EROWS + r0, _GROUP)], src_v)
            pltpu.sync_copy(dst_hbm.at[pl.ds(r0, _GROUP)], dst_v)
            for j in range(_GROUP):
                pltpu.async_copy(xs_hbm.at[src_v.at[j]], rows_v, sem).wait()
                pltpu.sync_copy(rows_v, acc_sh.at[dst_v.at[j]], add=True)
            return carry

        lax.fori_loop(0, slabs_per_tile // _GROUP, group, 0)
        plsc.subcore_barrier()
        r0 = s * 640
        pltpu.sync_copy(acc_sh.at[pl.ds(r0, 640)],
                        out_hbm.at[pl.ds(c * _N_ACC + r0, 640)])

    return k(xs, src2, dst, z128)


def _sc_spmm_narrow(tbl3, src, dst, z640):
    @functools.partial(
        pl.kernel,
        out_type=jax.ShapeDtypeStruct((_NC * 3 * _N_ACC,), jnp.float32),
        mesh=_sc_mesh(),
        scratch_types=[
            pltpu.VMEM((_GROUP, _SLAB), jnp.int32),
            pltpu.VMEM((_GROUP, _SLAB), jnp.int32),
            pltpu.VMEM((_SLAB,), jnp.float32),
            pltpu.VMEM((640,), jnp.float32),
            pltpu.VMEM_SHARED((3 * _N_ACC,), jnp.float32),
            pltpu.SemaphoreType.DMA,
        ],
    )
    def k(tbl_hbm, src_hbm, dst_hbm, z_hbm, out_hbm,
          src_v, dst_v, vals_v, bnc_v, acc_sh, sem):
        c = lax.axis_index("c")
        s = lax.axis_index("s")
        pltpu.sync_copy(z_hbm, bnc_v)
        for kk in range(3):
            pltpu.sync_copy(bnc_v, acc_sh.at[pl.ds(kk * _N_ACC + s * 640, 640)])
        plsc.subcore_barrier()

        rows_per_core = _EROWS // _NC
        rows_per_tile = rows_per_core // _NS

        def group(g, carry):
            r0 = c * rows_per_core + s * rows_per_tile + g * _GROUP
            pltpu.sync_copy(src_hbm.at[pl.ds(r0, _GROUP)], src_v)
            pltpu.sync_copy(dst_hbm.at[pl.ds(r0, _GROUP)], dst_v)
            for j in range(_GROUP):
                for kk in range(3):
                    pltpu.async_copy(
                        tbl_hbm.at[pl.ds(kk * _N, _N)].at[src_v.at[j]],
                        vals_v, sem).wait()
                    pltpu.sync_copy(
                        vals_v,
                        acc_sh.at[pl.ds(kk * _N_ACC, _N_ACC)].at[dst_v.at[j]],
                        add=True)
            return carry

        lax.fori_loop(0, rows_per_tile // _GROUP, group, 0)
        plsc.subcore_barrier()
        for kk in range(3):
            r0 = kk * _N_ACC + s * 640
            pltpu.sync_copy(acc_sh.at[pl.ds(r0, 640)],
                            out_hbm.at[pl.ds(c * 3 * _N_ACC + r0, 640)])

    return k(tbl3, src, dst, z640)


def _dot(a, b):
    return jnp.dot(a, b, preferred_element_type=jnp.float32)


def _dot_hi(a, b):
    return jnp.dot(a, b, precision=jax.lax.Precision.HIGHEST,
                   preferred_element_type=jnp.float32)


def _tc_encode(xs, we_stacked):
    nb, br = 20, _N // 10

    def body(x_r, w_r, o_r):
        o_r[...] = _dot(x_r[...], w_r[0])

    return pl.pallas_call(
        body,
        grid=(nb,),
        in_specs=[pl.BlockSpec((br, _D), lambda i: (i, 0)),
                  pl.BlockSpec((1, _D, _D), lambda i: (i // 10, 0, 0))],
        out_specs=pl.BlockSpec((br, _D), lambda i: (i, 0)),
        out_shape=jax.ShapeDtypeStruct((2 * _N, _D), jnp.float32),
    )(xs, we_stacked)


def _tc_dense(h1, h2, m1w, m1b, l1w, l1b, m2w, m2b, l2w, l2b,
              muw, mub, fw16, fb16):
    nb, br = 10, _N // 10

    def body(h1_r, h2_r, m1w_r, m1b_r, l1w_r, l1b_r,
             m2w_r, m2b_r, l2w_r, l2b_r, muw_r, mub_r, fw_r, fb_r,
             mu1_o, lv1_o, mu2_o, lv2_o, o16_o):
        g1 = h1_r[...]
        g2 = h2_r[...]
        mu1 = _dot(g1, m1w_r[...]) + m1b_r[...]
        lv1 = _dot(g1, l1w_r[...]) + l1b_r[...]
        mu2 = _dot(g2, m2w_r[...]) + m2b_r[...]
        lv2 = _dot(g2, l2w_r[...]) + l2b_r[...]
        v1 = jnp.exp(lv1)
        v2 = jnp.exp(lv2)
        m = jnp.maximum(v1, v2)
        e1 = jnp.exp(v1 - m)
        e2 = jnp.exp(v2 - m)
        ef = 0.5 * (mu1 * e1 + mu2 * e2) / (e1 + e2)
        mu_f = _dot(ef, muw_r[...]) + mub_r[...]
        o16 = _dot(mu_f, fw_r[...]) + fb_r[...]
        mu1_o[...] = mu1
        lv1_o[...] = lv1
        mu2_o[...] = mu2
        lv2_o[...] = lv2
        o16_o[...] = o16

    blk = lambda r, cdim: pl.BlockSpec((r, cdim), lambda i: (i, 0))
    wblk = lambda r, cdim: pl.BlockSpec((r, cdim), lambda i: (0, 0))
    return pl.pallas_call(
        body,
        grid=(nb,),
        in_specs=[blk(br, _D), blk(br, _D),
                  wblk(_D, _D), wblk(1, _D), wblk(_D, _D), wblk(1, _D),
                  wblk(_D, _D), wblk(1, _D), wblk(_D, _D), wblk(1, _D),
                  wblk(_D, _D), wblk(1, _D), wblk(_D, 16), wblk(1, 16)],
        out_specs=[blk(br, _D), blk(br, _D), blk(br, _D), blk(br, _D),
                   blk(br, 16)],
        out_shape=[jax.ShapeDtypeStruct((_N, _D), jnp.float32)] * 4
        + [jax.ShapeDtypeStruct((_N, 16), jnp.float32)],
    )(h1, h2, m1w, m1b, l1w, l1b, m2w, m2b, l2w, l2b,
      muw, mub, fw16, fb16)


def _tc_recon(s0, s1, wd1p, wd2p):
    nb, br = 10, _N // 10

    def body(s0_r, s1_r, wd1_r, wd2_r, r1_o, r2_o):
        sp = s0_r[...] + s1_r[...]
        r1_o[...] = _dot_hi(sp, wd1_r[...])
        r2_o[...] = _dot_hi(sp, wd2_r[...])

    blk = lambda r, cdim: pl.BlockSpec((r, cdim), lambda i: (i, 0))
    wblk = lambda r, cdim: pl.BlockSpec((r, cdim), lambda i: (0, 0))
    return pl.pallas_call(
        body,
        grid=(nb,),
        in_specs=[blk(br, 16), blk(br, 16), wblk(16, _D), wblk(16, _D)],
        out_specs=[blk(br, _D), blk(br, _D)],
        out_shape=[jax.ShapeDtypeStruct((_N, _D), jnp.float32)] * 2,
    )(s0, s1, wd1p, wd2p)


def kernel(features_omics1, features_omics2, adj_spatial, adj_feature_omics1,
           adj_feature_omics2, W_enc1, W_enc2, fc_mu1_w, fc_mu1_b,
           fc_logvar1_w, fc_logvar1_b, fc_mu2_w, fc_mu2_b, fc_logvar2_w,
           fc_logvar2_b, mu_w, mu_b, logvar_w, logvar_b, ib_w, ib_b,
           fus_w, fus_b, W_dec1, W_dec2):
    f32 = jnp.float32
    src = adj_spatial[0].astype(jnp.int32)
    dst = adj_spatial[1].astype(jnp.int32)
    npad = _E_PAD - _E
    ar = jnp.arange(npad, dtype=jnp.int32)
    src_p = jnp.concatenate([src, ar % _N])
    dst_p = jnp.concatenate([dst, _N + ar % (_N_ACC - _N)])
    src_r = src_p.reshape(_EROWS, _SLAB)
    dst_r = dst_p.reshape(_EROWS, _SLAB)
    src2 = jnp.concatenate([src_r, src_r + _N], axis=0)
    xs = jnp.concatenate([features_omics1, features_omics2], axis=0)
    we = jnp.stack([W_enc1, W_enc2])
    xw = _tc_encode(xs, we)
    z128 = jnp.zeros((80, _D), f32)

    hs = _sc_spmm_wide(xw, src2, dst_r, z128)
    h1 = hs[:_N]
    h2 = hs[_N_ACC:_N_ACC + _N]

    fw16 = jnp.pad(fus_w, ((0, 0), (0, 13)))
    fb16 = jnp.pad(fus_b, (0, 13)).reshape(1, 16)
    mu1, lv1, mu2, lv2, o16 = _tc_dense(
        h1, h2,
        fc_mu1_w, fc_mu1_b.reshape(1, -1), fc_logvar1_w, fc_logvar1_b.reshape(1, -1),
        fc_mu2_w, fc_mu2_b.reshape(1, -1), fc_logvar2_w, fc_logvar2_b.reshape(1, -1),
        mu_w, mu_b.reshape(1, -1), fw16, fb16)

    tbl3 = o16[:, :3].T.reshape(3 * _N)
    z640 = jnp.zeros((640,), f32)
    sparts = _sc_spmm_narrow(tbl3, src_r, dst_r, z640)
    sp = sparts.reshape(_NC, 3, _N_ACC)
    s0 = jnp.pad(sp[0, :, :_N].T, ((0, 0), (0, 13)))
    s1 = jnp.pad(sp[1, :, :_N].T, ((0, 0), (0, 13)))

    wd1p = jnp.pad(W_dec1, ((0, 13), (0, 0)))
    wd2p = jnp.pad(W_dec2, ((0, 13), (0, 0)))
    r1, r2 = _tc_recon(s0, s1, wd1p, wd2p)
    return (mu1, lv1, mu2, lv2, r1, r2)
